# baseline (speedup 1.0000x reference)
import functools

import jax
import jax.numpy as jnp
from jax import lax
from jax.experimental import pallas as pl
from jax.experimental.pallas import tpu as pltpu
from jax.experimental.pallas import tpu_sc as plsc

N = 10000
NP = 10112
DUMP = 10000
DH = 128
E = 320000
G = 512
NC, NS = 2, 16
NW = NC * NS
CHUNK = 128
VPC = CHUNK // 16
CPW0 = 32
CPW1 = 128
CPWMAX = max(CPW0, CPW1)
E0 = NS * CPW0 * CHUNK
E1 = NS * CPW1 * CHUNK
EP = E0 + E1
NBUF = 2
RPS = NP // NS
BLK = 1264
GRID = NP // BLK

DEG_CHUNK = 128
DEG_CPW = EP // (NW * DEG_CHUNK)


def _sc_worker_ids():
    cid = lax.axis_index("c")
    sid = lax.axis_index("s")
    return cid, sid


def _sc_deg_body(dst_hbm, ones_hbm, zeros_hbm, out_hbm, dst_v, ones_v, acc):
    cid, sid = _sc_worker_ids()
    wid = sid * NC + cid
    pltpu.sync_copy(zeros_hbm.at[pl.ds(sid * RPS, RPS)],
                    acc.at[pl.ds(sid * RPS, RPS)])
    pltpu.sync_copy(ones_hbm, ones_v)
    pltpu.sync_copy(dst_hbm.at[wid], dst_v)
    plsc.subcore_barrier()

    @pl.loop(0, DEG_CPW)
    def _(j):
        pltpu.sync_copy(ones_v, acc.at[dst_v.at[j]], add=True)

    plsc.subcore_barrier()
    pltpu.sync_copy(acc.at[pl.ds(sid * RPS, RPS)],
                    out_hbm.at[cid, pl.ds(sid * RPS, RPS)])


_sc_deg = functools.partial(
    pl.kernel,
    out_type=jax.ShapeDtypeStruct((NC, NP, DH), jnp.float32),
    mesh=plsc.VectorSubcoreMesh(core_axis_name="c", subcore_axis_name="s"),
    scratch_types=[
        pltpu.VMEM((DEG_CPW, DEG_CHUNK), jnp.int32),
        pltpu.VMEM((DEG_CHUNK, DH), jnp.float32),
        pltpu.VMEM_SHARED((NP, DH), jnp.float32),
    ],
)(_sc_deg_body)


def _sc_agg_body(s_hbm, packed_hbm, zeros_hbm, out_hbm,
                 pk_v, si0, si1, r0, r1, dring, g0, g1, acc):
    rows = (r0, r1)
    gsem = (g0, g1)
    srcidx = (si0, si1)
    cid, sid = _sc_worker_ids()
    ncw = CPW0 + cid * (CPW1 - CPW0)
    jmax = ncw // NBUF - 1
    pltpu.sync_copy(zeros_hbm.at[pl.ds(sid * RPS, RPS)],
                    acc.at[pl.ds(sid * RPS, RPS)])
    pltpu.sync_copy(packed_hbm.at[cid, sid], pk_v)
    plsc.subcore_barrier()

    def unpack(c_row, half, b, gslot):
        for i in range(VPC):
            off = CHUNK * half + 16 * i
            v = pk_v[c_row, pl.ds(off, 16)]
            srcidx[b][pl.ds(16 * i, 16)] = lax.bitwise_and(v, 16383)
            dring[gslot, b, pl.ds(16 * i, 16)] = lax.shift_right_logical(v, 14)

    for k in range(NBUF):
        unpack(k // 2, k % 2, k, 0)
        pltpu.async_copy(s_hbm.at[srcidx[k]], rows[k], gsem[k])

    @pl.loop(0, jmax)
    def _(jj):
        gcur = lax.rem(jj, 2)
        gnxt = lax.rem(jj + 1, 2)
        for k in range(NBUF):
            pltpu.make_async_copy(s_hbm.at[srcidx[k]], rows[k],
                                  gsem[k]).wait()
            pltpu.sync_copy(rows[k], acc.at[dring.at[gcur, k]], add=True)
            unpack(jj + 1, k, k, gnxt)
            pltpu.async_copy(s_hbm.at[srcidx[k]], rows[k], gsem[k])

    glast = lax.rem(jmax, 2)
    for k in range(NBUF):
        pltpu.make_async_copy(s_hbm.at[srcidx[k]], rows[k], gsem[k]).wait()
        pltpu.sync_copy(rows[k], acc.at[dring.at[glast, k]], add=True)

    plsc.subcore_barrier()
    pltpu.sync_copy(acc.at[pl.ds(sid * RPS, RPS)],
                    out_hbm.at[cid, pl.ds(sid * RPS, RPS)])


_sc_agg = functools.partial(
    pl.kernel,
    out_type=jax.ShapeDtypeStruct((NC, NP, DH), jnp.float32),
    mesh=plsc.VectorSubcoreMesh(core_axis_name="c", subcore_axis_name="s"),
    scratch_types=[
        pltpu.VMEM((CPWMAX // 2, CHUNK * 2), jnp.int32),
        pltpu.VMEM((CHUNK,), jnp.int32),
        pltpu.VMEM((CHUNK,), jnp.int32),
        pltpu.VMEM((CHUNK, DH), jnp.float32),
        pltpu.VMEM((CHUNK, DH), jnp.float32),
        pltpu.VMEM((2, NBUF, CHUNK), jnp.int32),
        pltpu.SemaphoreType.DMA,
        pltpu.SemaphoreType.DMA,
        pltpu.VMEM_SHARED((NP, DH), jnp.float32),
    ],
)(_sc_agg_body)


def _tc1_body(x_ref, d_ref, w1_ref, s1_ref, dv_ref):
    deg = 1.0 + d_ref[0, :, 0:1] + d_ref[1, :, 0:1]
    dv = lax.rsqrt(deg)
    h = jnp.dot(x_ref[...], w1_ref[...], preferred_element_type=jnp.float32)
    s1_ref[...] = h * dv
    dv_ref[...] = jnp.broadcast_to(dv, (BLK, DH))


def _tc1(x_p, deg_part, w1):
    return pl.pallas_call(
        _tc1_body,
        grid=(GRID,),
        in_specs=[
            pl.BlockSpec((BLK, DH), lambda i: (i, 0)),
            pl.BlockSpec((NC, BLK, DH), lambda i: (0, i, 0)),
            pl.BlockSpec((DH, DH), lambda i: (0, 0)),
        ],
        out_specs=[
            pl.BlockSpec((BLK, DH), lambda i: (i, 0)),
            pl.BlockSpec((BLK, DH), lambda i: (i, 0)),
        ],
        out_shape=[
            jax.ShapeDtypeStruct((NP, DH), jnp.float32),
            jax.ShapeDtypeStruct((NP, DH), jnp.float32),
        ],
    )(x_p, deg_part, w1)


def _tc2_body(agg_ref, s1_ref, dv_ref, b1_ref, w2_ref, s2_ref):
    t = (agg_ref[0] + agg_ref[1] + s1_ref[...]) * dv_ref[...] + b1_ref[...]
    t = jnp.maximum(t, 0.0)
    s2_ref[...] = jnp.dot(
        t, w2_ref[...], preferred_element_type=jnp.float32) * dv_ref[...]


def _tc2(agg1, s1, dv, b1, w2):
    return pl.pallas_call(
        _tc2_body,
        grid=(GRID,),
        in_specs=[
            pl.BlockSpec((NC, BLK, DH), lambda i: (0, i, 0)),
            pl.BlockSpec((BLK, DH), lambda i: (i, 0)),
            pl.BlockSpec((BLK, DH), lambda i: (i, 0)),
            pl.BlockSpec((1, DH), lambda i: (0, 0)),
            pl.BlockSpec((DH, DH), lambda i: (0, 0)),
        ],
        out_specs=pl.BlockSpec((BLK, DH), lambda i: (i, 0)),
        out_shape=jax.ShapeDtypeStruct((NP, DH), jnp.float32),
    )(agg1, s1, dv, b1, w2)


def _tc3_body(agg_ref, s2_ref, dv_ref, b2_ref, batch_ref,
              f1w_ref, f1b_ref, f2w_ref, f2b_ref, out_ref,
              sums_acc, cnt_acc):
    i = pl.program_id(0)

    @pl.when(i == 0)
    def _():
        sums_acc[...] = jnp.zeros_like(sums_acc)
        cnt_acc[...] = jnp.zeros_like(cnt_acc)

    t2 = (agg_ref[0] + agg_ref[1] + s2_ref[...]) * dv_ref[...] + b2_ref[...]
    t2 = jnp.maximum(t2, 0.0)
    ids = batch_ref[0, 0, :]
    gids = lax.broadcasted_iota(jnp.int32, (G, BLK), 0)
    onehot = (ids[None, :] == gids).astype(jnp.float32)
    sums_acc[...] += jnp.dot(onehot, t2, preferred_element_type=jnp.float32)
    cnt_acc[...] += jnp.sum(onehot, axis=1, keepdims=True)

    @pl.when(i == GRID - 1)
    def _():
        pooled = sums_acc[...] / jnp.maximum(cnt_acc[...], 1.0)
        r = jnp.dot(pooled, f1w_ref[...],
                    preferred_element_type=jnp.float32) + f1b_ref[...]
        r = jnp.maximum(r, 0.0)
        out_ref[...] = jnp.dot(
            r, f2w_ref[...], preferred_element_type=jnp.float32) + f2b_ref[...]


def _tc3(agg2, s2, dv, b2, batch_p, f1w, f1b, f2w, f2b):
    return pl.pallas_call(
        _tc3_body,
        grid=(GRID,),
        in_specs=[
            pl.BlockSpec((NC, BLK, DH), lambda i: (0, i, 0)),
            pl.BlockSpec((BLK, DH), lambda i: (i, 0)),
            pl.BlockSpec((BLK, DH), lambda i: (i, 0)),
            pl.BlockSpec((1, DH), lambda i: (0, 0)),
            pl.BlockSpec((1, 1, BLK), lambda i: (i, 0, 0)),
            pl.BlockSpec((DH, DH), lambda i: (0, 0)),
            pl.BlockSpec((1, DH), lambda i: (0, 0)),
            pl.BlockSpec((DH, 1), lambda i: (0, 0)),
            pl.BlockSpec((1, 1), lambda i: (0, 0)),
        ],
        out_specs=pl.BlockSpec((G, 1), lambda i: (0, 0)),
        out_shape=jax.ShapeDtypeStruct((G, 1), jnp.float32),
        scratch_shapes=[
            pltpu.VMEM((G, DH), jnp.float32),
            pltpu.VMEM((G, 1), jnp.float32),
        ],
    )(agg2, s2, dv, b2, batch_p, f1w, f1b, f2w, f2b)


def _pack_edges(src, dst):
    src_pad = jnp.concatenate([src, jnp.zeros((EP - E,), jnp.int32)])
    dst_pad = jnp.concatenate([dst, jnp.full((EP - E,), DUMP, jnp.int32)])
    pk = src_pad | (dst_pad << 14)
    pk0 = pk[:E0].reshape(NS, CPW0 // 2, CHUNK * 2)
    pk0 = jnp.pad(pk0, ((0, 0), (0, (CPWMAX - CPW0) // 2), (0, 0)))
    pk1 = pk[E0:].reshape(NS, CPW1 // 2, CHUNK * 2)
    pk1 = jnp.pad(pk1, ((0, 0), (0, (CPWMAX - CPW1) // 2), (0, 0)))
    packed = jnp.stack([pk0, pk1])
    return dst_pad, packed


def kernel(x, edge_index, batch, W1, b1, W2, b2, fc1_W, fc1_b, fc2_W, fc2_b):
    f32 = jnp.float32
    src = edge_index[0].astype(jnp.int32)
    dst = edge_index[1].astype(jnp.int32)
    dst_pad, packed = _pack_edges(src, dst)
    dst_p = dst_pad.reshape(NW, DEG_CPW, DEG_CHUNK)
    x_p = jnp.pad(x.astype(f32), ((0, NP - N), (0, 0)))
    batch_p = jnp.concatenate(
        [batch.astype(jnp.int32),
         jnp.full((NP - N,), G, jnp.int32)]).reshape(GRID, 1, BLK)
    zeros128 = jnp.zeros((NP, DH), f32)
    ones128 = jnp.ones((DEG_CHUNK, DH), f32)

    deg_part = _sc_deg(dst_p, ones128, zeros128)
    s1, dv = _tc1(x_p, deg_part, W1)
    agg1 = _sc_agg(s1, packed, zeros128)
    s2 = _tc2(agg1, s1, dv, b1.reshape(1, DH), W2)
    agg2 = _sc_agg(s2, packed, zeros128)
    return _tc3(agg2, s2, dv, b2.reshape(1, DH), batch_p,
                fc1_W, fc1_b.reshape(1, DH), fc2_W, fc2_b.reshape(1, 1))


# speedup vs baseline: 1.1692x; 1.1692x over previous
import functools

import jax
import jax.numpy as jnp
from jax import lax
from jax.experimental import pallas as pl
from jax.experimental.pallas import tpu as pltpu
from jax.experimental.pallas import tpu_sc as plsc

N = 10000
NP = 10112
DUMP = 10000
DH = 128
E = 320000
G = 512
NC, NS = 2, 16
NW = NC * NS
CHUNK = 128
VPC = CHUNK // 16
CPW0 = 128
CPW1 = 32
CPWMAX = max(CPW0, CPW1)
E0 = NS * CPW0 * CHUNK
E1 = NS * CPW1 * CHUNK
EP = E0 + E1
NBUF = 2
RPS = NP // NS
BLK = 1264
GRID = NP // BLK

DEG_CHUNK = 128
DEG_CPW = EP // (NW * DEG_CHUNK)


def _sc_worker_ids():
    cid = lax.axis_index("c")
    sid = lax.axis_index("s")
    return cid, sid


def _sc_deg_body(dst_hbm, ones_hbm, zeros_hbm, out_hbm, dst_v, ones_v, acc):
    cid, sid = _sc_worker_ids()
    wid = sid * NC + cid
    pltpu.sync_copy(zeros_hbm.at[pl.ds(sid * RPS, RPS)],
                    acc.at[pl.ds(sid * RPS, RPS)])
    pltpu.sync_copy(ones_hbm, ones_v)
    pltpu.sync_copy(dst_hbm.at[wid], dst_v)
    plsc.subcore_barrier()

    @pl.loop(0, DEG_CPW)
    def _(j):
        pltpu.sync_copy(ones_v, acc.at[dst_v.at[j]], add=True)

    plsc.subcore_barrier()
    pltpu.sync_copy(acc.at[pl.ds(sid * RPS, RPS)],
                    out_hbm.at[cid, pl.ds(sid * RPS, RPS)])


_sc_deg = functools.partial(
    pl.kernel,
    out_type=jax.ShapeDtypeStruct((NC, NP, DH), jnp.float32),
    mesh=plsc.VectorSubcoreMesh(core_axis_name="c", subcore_axis_name="s"),
    scratch_types=[
        pltpu.VMEM((DEG_CPW, DEG_CHUNK), jnp.int32),
        pltpu.VMEM((DEG_CHUNK, DH), jnp.float32),
        pltpu.VMEM_SHARED((NP, DH), jnp.float32),
    ],
)(_sc_deg_body)


def _sc_agg_body(s_hbm, packed_hbm, zeros_hbm, out_hbm,
                 pk_v, si0, si1, r0, r1, dring, g0, g1, acc):
    rows = (r0, r1)
    gsem = (g0, g1)
    srcidx = (si0, si1)
    cid, sid = _sc_worker_ids()
    ncw = CPW0 + cid * (CPW1 - CPW0)
    jmax = ncw // NBUF - 1
    pltpu.sync_copy(zeros_hbm.at[pl.ds(sid * RPS, RPS)],
                    acc.at[pl.ds(sid * RPS, RPS)])
    pltpu.sync_copy(packed_hbm.at[cid, sid], pk_v)
    plsc.subcore_barrier()

    def unpack(c_row, half, b, gslot):
        for i in range(VPC):
            off = CHUNK * half + 16 * i
            v = pk_v[c_row, pl.ds(off, 16)]
            srcidx[b][pl.ds(16 * i, 16)] = lax.bitwise_and(v, 16383)
            dring[gslot, b, pl.ds(16 * i, 16)] = lax.shift_right_logical(v, 14)

    for k in range(NBUF):
        unpack(k // 2, k % 2, k, 0)
        pltpu.async_copy(s_hbm.at[srcidx[k]], rows[k], gsem[k])

    @pl.loop(0, jmax)
    def _(jj):
        gcur = lax.rem(jj, 2)
        gnxt = lax.rem(jj + 1, 2)
        for k in range(NBUF):
            pltpu.make_async_copy(s_hbm.at[srcidx[k]], rows[k],
                                  gsem[k]).wait()
            pltpu.sync_copy(rows[k], acc.at[dring.at[gcur, k]], add=True)
            unpack(jj + 1, k, k, gnxt)
            pltpu.async_copy(s_hbm.at[srcidx[k]], rows[k], gsem[k])

    glast = lax.rem(jmax, 2)
    for k in range(NBUF):
        pltpu.make_async_copy(s_hbm.at[srcidx[k]], rows[k], gsem[k]).wait()
        pltpu.sync_copy(rows[k], acc.at[dring.at[glast, k]], add=True)

    plsc.subcore_barrier()
    pltpu.sync_copy(acc.at[pl.ds(sid * RPS, RPS)],
                    out_hbm.at[cid, pl.ds(sid * RPS, RPS)])


_sc_agg = functools.partial(
    pl.kernel,
    out_type=jax.ShapeDtypeStruct((NC, NP, DH), jnp.float32),
    mesh=plsc.VectorSubcoreMesh(core_axis_name="c", subcore_axis_name="s"),
    scratch_types=[
        pltpu.VMEM((CPWMAX // 2, CHUNK * 2), jnp.int32),
        pltpu.VMEM((CHUNK,), jnp.int32),
        pltpu.VMEM((CHUNK,), jnp.int32),
        pltpu.VMEM((CHUNK, DH), jnp.float32),
        pltpu.VMEM((CHUNK, DH), jnp.float32),
        pltpu.VMEM((2, NBUF, CHUNK), jnp.int32),
        pltpu.SemaphoreType.DMA,
        pltpu.SemaphoreType.DMA,
        pltpu.VMEM_SHARED((NP, DH), jnp.float32),
    ],
)(_sc_agg_body)


def _tc1_body(x_ref, d_ref, w1_ref, s1_ref, dv_ref):
    deg = 1.0 + d_ref[0, :, 0:1] + d_ref[1, :, 0:1]
    dv = lax.rsqrt(deg)
    h = jnp.dot(x_ref[...], w1_ref[...], preferred_element_type=jnp.float32)
    s1_ref[...] = h * dv
    dv_ref[...] = jnp.broadcast_to(dv, (BLK, DH))


def _tc1(x_p, deg_part, w1):
    return pl.pallas_call(
        _tc1_body,
        grid=(GRID,),
        in_specs=[
            pl.BlockSpec((BLK, DH), lambda i: (i, 0)),
            pl.BlockSpec((NC, BLK, DH), lambda i: (0, i, 0)),
            pl.BlockSpec((DH, DH), lambda i: (0, 0)),
        ],
        out_specs=[
            pl.BlockSpec((BLK, DH), lambda i: (i, 0)),
            pl.BlockSpec((BLK, DH), lambda i: (i, 0)),
        ],
        out_shape=[
            jax.ShapeDtypeStruct((NP, DH), jnp.float32),
            jax.ShapeDtypeStruct((NP, DH), jnp.float32),
        ],
    )(x_p, deg_part, w1)


def _tc2_body(agg_ref, s1_ref, dv_ref, b1_ref, w2_ref, s2_ref):
    t = (agg_ref[0] + agg_ref[1] + s1_ref[...]) * dv_ref[...] + b1_ref[...]
    t = jnp.maximum(t, 0.0)
    s2_ref[...] = jnp.dot(
        t, w2_ref[...], preferred_element_type=jnp.float32) * dv_ref[...]


def _tc2(agg1, s1, dv, b1, w2):
    return pl.pallas_call(
        _tc2_body,
        grid=(GRID,),
        in_specs=[
            pl.BlockSpec((NC, BLK, DH), lambda i: (0, i, 0)),
            pl.BlockSpec((BLK, DH), lambda i: (i, 0)),
            pl.BlockSpec((BLK, DH), lambda i: (i, 0)),
            pl.BlockSpec((1, DH), lambda i: (0, 0)),
            pl.BlockSpec((DH, DH), lambda i: (0, 0)),
        ],
        out_specs=pl.BlockSpec((BLK, DH), lambda i: (i, 0)),
        out_shape=jax.ShapeDtypeStruct((NP, DH), jnp.float32),
    )(agg1, s1, dv, b1, w2)


def _tc3_body(agg_ref, s2_ref, dv_ref, b2_ref, batch_ref,
              f1w_ref, f1b_ref, f2w_ref, f2b_ref, out_ref,
              sums_acc, cnt_acc):
    i = pl.program_id(0)

    @pl.when(i == 0)
    def _():
        sums_acc[...] = jnp.zeros_like(sums_acc)
        cnt_acc[...] = jnp.zeros_like(cnt_acc)

    t2 = (agg_ref[0] + agg_ref[1] + s2_ref[...]) * dv_ref[...] + b2_ref[...]
    t2 = jnp.maximum(t2, 0.0)
    ids = batch_ref[0, 0, :]
    gids = lax.broadcasted_iota(jnp.int32, (G, BLK), 0)
    onehot = (ids[None, :] == gids).astype(jnp.float32)
    sums_acc[...] += jnp.dot(onehot, t2, preferred_element_type=jnp.float32)
    cnt_acc[...] += jnp.sum(onehot, axis=1, keepdims=True)

    @pl.when(i == GRID - 1)
    def _():
        pooled = sums_acc[...] / jnp.maximum(cnt_acc[...], 1.0)
        r = jnp.dot(pooled, f1w_ref[...],
                    preferred_element_type=jnp.float32) + f1b_ref[...]
        r = jnp.maximum(r, 0.0)
        out_ref[...] = jnp.dot(
            r, f2w_ref[...], preferred_element_type=jnp.float32) + f2b_ref[...]


def _tc3(agg2, s2, dv, b2, batch_p, f1w, f1b, f2w, f2b):
    return pl.pallas_call(
        _tc3_body,
        grid=(GRID,),
        in_specs=[
            pl.BlockSpec((NC, BLK, DH), lambda i: (0, i, 0)),
            pl.BlockSpec((BLK, DH), lambda i: (i, 0)),
            pl.BlockSpec((BLK, DH), lambda i: (i, 0)),
            pl.BlockSpec((1, DH), lambda i: (0, 0)),
            pl.BlockSpec((1, 1, BLK), lambda i: (i, 0, 0)),
            pl.BlockSpec((DH, DH), lambda i: (0, 0)),
            pl.BlockSpec((1, DH), lambda i: (0, 0)),
            pl.BlockSpec((DH, 1), lambda i: (0, 0)),
            pl.BlockSpec((1, 1), lambda i: (0, 0)),
        ],
        out_specs=pl.BlockSpec((G, 1), lambda i: (0, 0)),
        out_shape=jax.ShapeDtypeStruct((G, 1), jnp.float32),
        scratch_shapes=[
            pltpu.VMEM((G, DH), jnp.float32),
            pltpu.VMEM((G, 1), jnp.float32),
        ],
    )(agg2, s2, dv, b2, batch_p, f1w, f1b, f2w, f2b)


def _pack_edges(src, dst):
    src_pad = jnp.concatenate([src, jnp.zeros((EP - E,), jnp.int32)])
    dst_pad = jnp.concatenate([dst, jnp.full((EP - E,), DUMP, jnp.int32)])
    pk = src_pad | (dst_pad << 14)
    pk0 = pk[:E0].reshape(NS, CPW0 // 2, CHUNK * 2)
    pk0 = jnp.pad(pk0, ((0, 0), (0, (CPWMAX - CPW0) // 2), (0, 0)))
    pk1 = pk[E0:].reshape(NS, CPW1 // 2, CHUNK * 2)
    pk1 = jnp.pad(pk1, ((0, 0), (0, (CPWMAX - CPW1) // 2), (0, 0)))
    packed = jnp.stack([pk0, pk1])
    return dst_pad, packed


def kernel(x, edge_index, batch, W1, b1, W2, b2, fc1_W, fc1_b, fc2_W, fc2_b):
    f32 = jnp.float32
    src = edge_index[0].astype(jnp.int32)
    dst = edge_index[1].astype(jnp.int32)
    dst_pad, packed = _pack_edges(src, dst)
    dst_p = dst_pad.reshape(NW, DEG_CPW, DEG_CHUNK)
    x_p = jnp.pad(x.astype(f32), ((0, NP - N), (0, 0)))
    batch_p = jnp.concatenate(
        [batch.astype(jnp.int32),
         jnp.full((NP - N,), G, jnp.int32)]).reshape(GRID, 1, BLK)
    zeros128 = jnp.zeros((NP, DH), f32)
    ones128 = jnp.ones((DEG_CHUNK, DH), f32)

    deg_part = _sc_deg(dst_p, ones128, zeros128)
    s1, dv = _tc1(x_p, deg_part, W1)
    agg1 = _sc_agg(s1, packed, zeros128)
    s2 = _tc2(agg1, s1, dv, b1.reshape(1, DH), W2)
    agg2 = _sc_agg(s2, packed, zeros128)
    return _tc3(agg2, s2, dv, b2.reshape(1, DH), batch_p,
                fc1_W, fc1_b.reshape(1, DH), fc2_W, fc2_b.reshape(1, 1))
